# baseline (speedup 1.0000x reference)
import functools

import jax
import jax.numpy as jnp
from jax import lax
from jax.experimental import pallas as pl
from jax.experimental.pallas import tpu as pltpu
from jax.experimental.pallas import tpu_sc as plsc

_NC = 2
_NS = 16
_NW = _NC * _NS
_C = 128
_NBUF = 5
_AHEAD = 2


@functools.lru_cache(maxsize=None)
def _make_lookup(b_total: int, d: int):
    bpw = b_total // _NW
    nchunk = bpw // _C
    mesh = plsc.VectorSubcoreMesh(
        core_axis_name="c", subcore_axis_name="s",
        num_cores=_NC, num_subcores=_NS,
    )

    @functools.partial(
        pl.kernel,
        out_type=jax.ShapeDtypeStruct((b_total, d), jnp.float32),
        mesh=mesh,
        scratch_types=[pltpu.VMEM((nchunk, _C), jnp.int32)]
        + [pltpu.VMEM((_C, d), jnp.float32)] * _NBUF
        + [pltpu.SemaphoreType.DMA] * (2 * _NBUF),
    )
    def lookup(table_hbm, idx_hbm, out_hbm, idx_v, *rest):
        bufs = rest[:_NBUF]
        gsem = rest[_NBUF:2 * _NBUF]
        ssem = rest[2 * _NBUF:]
        wid = lax.axis_index("s") * _NC + lax.axis_index("c")
        base = wid * bpw
        pltpu.sync_copy(idx_hbm.at[wid], idx_v)

        for c in range(_AHEAD):
            pltpu.async_copy(table_hbm.at[idx_v.at[c]], bufs[c], gsem[c])

        @pl.loop(0, nchunk, step=_NBUF)
        def _(g):
            for u in range(_NBUF):
                c = g + u
                bj = (u + _AHEAD) % _NBUF
                j = c + _AHEAD

                @pl.when(j < nchunk)
                def _():
                    @pl.when(j >= _NBUF)
                    def _():
                        pltpu.make_async_copy(
                            bufs[bj],
                            out_hbm.at[pl.ds(base + (j - _NBUF) * _C, _C)],
                            ssem[bj]).wait()

                    pltpu.async_copy(
                        table_hbm.at[idx_v.at[j]], bufs[bj], gsem[bj])

                pltpu.make_async_copy(
                    table_hbm.at[idx_v.at[c]], bufs[u], gsem[u]).wait()
                pltpu.async_copy(
                    bufs[u], out_hbm.at[pl.ds(base + c * _C, _C)], ssem[u])

        for u in range(_NBUF):
            c_last = nchunk - _NBUF + u
            pltpu.make_async_copy(
                bufs[u], out_hbm.at[pl.ds(base + c_last * _C, _C)],
                ssem[u]).wait()

    return lookup


def kernel(input_ids, weight):
    orig_shape = input_ids.shape
    d = weight.shape[1]
    if input_ids.dtype != jnp.int32:
        input_ids = input_ids.astype(jnp.int32)
    flat = jnp.reshape(input_ids, (-1,))
    b = flat.shape[0]
    blk = _NW * _C * _NBUF
    b_pad = ((b + blk - 1) // blk) * blk
    if b_pad != b:
        flat = jnp.concatenate(
            [flat, jnp.zeros((b_pad - b,), jnp.int32)])
    tc_rows = blk
    sc_b = b_pad - tc_rows
    idx = jnp.reshape(flat[:sc_b], (_NW, sc_b // (_NW * _C), _C))
    sc_out = _make_lookup(sc_b, d)(weight.astype(jnp.float32), idx)
    tc_out = jnp.take(weight.astype(jnp.float32), flat[sc_b:], axis=0)
    out = jnp.concatenate([sc_out, tc_out], axis=0)
    if b_pad != b:
        out = out[:b]
    return jnp.reshape(out, orig_shape + (d,))


# speedup vs baseline: 1.8094x; 1.8094x over previous
import functools

import jax
import jax.numpy as jnp
from jax import lax
from jax.experimental import pallas as pl
from jax.experimental.pallas import tpu as pltpu
from jax.experimental.pallas import tpu_sc as plsc

_NC = 2
_NS = 16
_NW = _NC * _NS
_C = 128
_NBUF = 4
_AHEAD = 2


def _col_parts(k: int):
    return [(off, min(_C, k - off)) for off in range(0, k, _C)]


@functools.lru_cache(maxsize=None)
def _make_lookup_2d(r_total: int, k: int, d: int):
    rw = r_total // _NW
    parts = _col_parts(k)
    npart = len(parts)
    units = rw * npart
    mesh = plsc.VectorSubcoreMesh(
        core_axis_name="c", subcore_axis_name="s",
        num_cores=_NC, num_subcores=_NS,
    )

    @functools.partial(
        pl.kernel,
        out_type=jax.ShapeDtypeStruct((r_total * k, d), jnp.float32),
        mesh=mesh,
        scratch_types=[pltpu.VMEM((rw, k), jnp.int32)]
        + [pltpu.VMEM((parts[u % npart][1], d), jnp.float32)
           for u in range(_NBUF)]
        + [pltpu.SemaphoreType.DMA] * (2 * _NBUF),
    )
    def lookup(ids_hbm, table_hbm, out_hbm, idx_v, *rest):
        bufs = rest[:_NBUF]
        gsem = rest[_NBUF:2 * _NBUF]
        ssem = rest[2 * _NBUF:]
        wid = lax.axis_index("s") * _NC + lax.axis_index("c")
        base = wid * rw * k
        pltpu.sync_copy(ids_hbm.at[pl.ds(wid * rw, rw)], idx_v)

        def gidx(u_off, g):
            off, w = parts[u_off % npart]
            return idx_v.at[(g + u_off) // npart, pl.ds(off, w)]

        def oslice(u_off, g):
            off, w = parts[u_off % npart]
            return out_hbm.at[
                pl.ds(base + ((g + u_off) // npart) * k + off, w)]

        for u in range(_AHEAD):
            pltpu.async_copy(table_hbm.at[gidx(u, 0)], bufs[u], gsem[u])

        @pl.loop(0, units, step=_NBUF)
        def _(g):
            for u in range(_NBUF):
                uj = u + _AHEAD
                bj = uj % _NBUF
                ju = g + uj

                @pl.when(ju < units)
                def _():
                    @pl.when(ju >= _NBUF)
                    def _():
                        pltpu.make_async_copy(
                            bufs[bj], oslice(uj - _NBUF, g), ssem[bj]).wait()

                    pltpu.async_copy(
                        table_hbm.at[gidx(uj, g)], bufs[bj], gsem[bj])

                pltpu.make_async_copy(
                    table_hbm.at[gidx(u, g)], bufs[u], gsem[u]).wait()
                pltpu.async_copy(bufs[u], oslice(u, g), ssem[u])

        for u in range(_NBUF):
            pltpu.make_async_copy(
                bufs[u], oslice(u, units - _NBUF), ssem[u]).wait()

    return lookup


@functools.lru_cache(maxsize=None)
def _make_lookup_flat(b_total: int, d: int):
    bpw = b_total // _NW
    nchunk = bpw // _C
    mesh = plsc.VectorSubcoreMesh(
        core_axis_name="c", subcore_axis_name="s",
        num_cores=_NC, num_subcores=_NS,
    )

    @functools.partial(
        pl.kernel,
        out_type=jax.ShapeDtypeStruct((b_total, d), jnp.float32),
        mesh=mesh,
        scratch_types=[pltpu.VMEM((nchunk, _C), jnp.int32)]
        + [pltpu.VMEM((_C, d), jnp.float32)] * _NBUF
        + [pltpu.SemaphoreType.DMA] * (2 * _NBUF),
    )
    def lookup(table_hbm, idx_hbm, out_hbm, idx_v, *rest):
        bufs = rest[:_NBUF]
        gsem = rest[_NBUF:2 * _NBUF]
        ssem = rest[2 * _NBUF:]
        wid = lax.axis_index("s") * _NC + lax.axis_index("c")
        base = wid * bpw
        pltpu.sync_copy(idx_hbm.at[wid], idx_v)

        for c in range(_AHEAD):
            pltpu.async_copy(table_hbm.at[idx_v.at[c]], bufs[c], gsem[c])

        @pl.loop(0, nchunk, step=_NBUF)
        def _(g):
            for u in range(_NBUF):
                c = g + u
                bj = (u + _AHEAD) % _NBUF
                j = c + _AHEAD

                @pl.when(j < nchunk)
                def _():
                    @pl.when(j >= _NBUF)
                    def _():
                        pltpu.make_async_copy(
                            bufs[bj],
                            out_hbm.at[pl.ds(base + (j - _NBUF) * _C, _C)],
                            ssem[bj]).wait()

                    pltpu.async_copy(
                        table_hbm.at[idx_v.at[j]], bufs[bj], gsem[bj])

                pltpu.make_async_copy(
                    table_hbm.at[idx_v.at[c]], bufs[u], gsem[u]).wait()
                pltpu.async_copy(
                    bufs[u], out_hbm.at[pl.ds(base + c * _C, _C)], ssem[u])

        for u in range(_NBUF):
            c_last = nchunk - _NBUF + u
            pltpu.make_async_copy(
                bufs[u], out_hbm.at[pl.ds(base + c_last * _C, _C)],
                ssem[u]).wait()

    return lookup


def kernel(input_ids, weight):
    orig_shape = input_ids.shape
    d = weight.shape[1]
    w32 = weight.astype(jnp.float32)
    if input_ids.dtype != jnp.int32:
        input_ids = input_ids.astype(jnp.int32)

    if len(orig_shape) == 2:
        r, k = orig_shape
        npart = len(_col_parts(k))
        if (r % _NW == 0 and k % 8 == 0 and _NBUF % npart == 0
                and ((r // _NW) * npart) % _NBUF == 0):
            out = _make_lookup_2d(r, k, d)(input_ids, w32)
            return jnp.reshape(out, orig_shape + (d,))

    flat = jnp.reshape(input_ids, (-1,))
    b = flat.shape[0]
    blk = _NW * _C * _NBUF
    b_pad = ((b + blk - 1) // blk) * blk
    if b_pad != b:
        flat = jnp.concatenate([flat, jnp.zeros((b_pad - b,), jnp.int32)])
    idx = jnp.reshape(flat, (_NW, b_pad // (_NW * _C), _C))
    out = _make_lookup_flat(b_pad, d)(w32, idx)
    if b_pad != b:
        out = out[:b]
    return jnp.reshape(out, orig_shape + (d,))


# speedup vs baseline: 1.8293x; 1.0110x over previous
import functools

import jax
import jax.numpy as jnp
from jax import lax
from jax.experimental import pallas as pl
from jax.experimental.pallas import tpu as pltpu
from jax.experimental.pallas import tpu_sc as plsc

_NC = 2
_NS = 16
_NW = _NC * _NS
_C = 128
_NBUF = 8
_AHEAD = 4


def _col_parts(k: int):
    return [(off, min(_C, k - off)) for off in range(0, k, _C)]


@functools.lru_cache(maxsize=None)
def _make_lookup_2d(r_total: int, k: int, d: int):
    rw = r_total // _NW
    parts = _col_parts(k)
    npart = len(parts)
    units = rw * npart
    mesh = plsc.VectorSubcoreMesh(
        core_axis_name="c", subcore_axis_name="s",
        num_cores=_NC, num_subcores=_NS,
    )

    @functools.partial(
        pl.kernel,
        out_type=jax.ShapeDtypeStruct((r_total * k, d), jnp.float32),
        mesh=mesh,
        scratch_types=[pltpu.VMEM((rw, k), jnp.int32)]
        + [pltpu.VMEM((parts[u % npart][1], d), jnp.float32)
           for u in range(_NBUF)]
        + [pltpu.SemaphoreType.DMA] * (2 * _NBUF),
    )
    def lookup(ids_hbm, table_hbm, out_hbm, idx_v, *rest):
        bufs = rest[:_NBUF]
        gsem = rest[_NBUF:2 * _NBUF]
        ssem = rest[2 * _NBUF:]
        wid = lax.axis_index("s") * _NC + lax.axis_index("c")
        base = wid * rw * k
        pltpu.sync_copy(ids_hbm.at[pl.ds(wid * rw, rw)], idx_v)

        def gidx(u_off, g):
            off, w = parts[u_off % npart]
            return idx_v.at[(g + u_off) // npart, pl.ds(off, w)]

        def oslice(u_off, g):
            off, w = parts[u_off % npart]
            return out_hbm.at[
                pl.ds(base + ((g + u_off) // npart) * k + off, w)]

        for u in range(_AHEAD):
            pltpu.async_copy(table_hbm.at[gidx(u, 0)], bufs[u], gsem[u])

        @pl.loop(0, units, step=_NBUF)
        def _(g):
            for u in range(_NBUF):
                uj = u + _AHEAD
                bj = uj % _NBUF
                ju = g + uj

                @pl.when(ju < units)
                def _():
                    @pl.when(ju >= _NBUF)
                    def _():
                        pltpu.make_async_copy(
                            bufs[bj], oslice(uj - _NBUF, g), ssem[bj]).wait()

                    pltpu.async_copy(
                        table_hbm.at[gidx(uj, g)], bufs[bj], gsem[bj])

                pltpu.make_async_copy(
                    table_hbm.at[gidx(u, g)], bufs[u], gsem[u]).wait()
                pltpu.async_copy(bufs[u], oslice(u, g), ssem[u])

        for u in range(_NBUF):
            pltpu.make_async_copy(
                bufs[u], oslice(u, units - _NBUF), ssem[u]).wait()

    return lookup


@functools.lru_cache(maxsize=None)
def _make_lookup_flat(b_total: int, d: int):
    bpw = b_total // _NW
    nchunk = bpw // _C
    mesh = plsc.VectorSubcoreMesh(
        core_axis_name="c", subcore_axis_name="s",
        num_cores=_NC, num_subcores=_NS,
    )

    @functools.partial(
        pl.kernel,
        out_type=jax.ShapeDtypeStruct((b_total, d), jnp.float32),
        mesh=mesh,
        scratch_types=[pltpu.VMEM((nchunk, _C), jnp.int32)]
        + [pltpu.VMEM((_C, d), jnp.float32)] * _NBUF
        + [pltpu.SemaphoreType.DMA] * (2 * _NBUF),
    )
    def lookup(table_hbm, idx_hbm, out_hbm, idx_v, *rest):
        bufs = rest[:_NBUF]
        gsem = rest[_NBUF:2 * _NBUF]
        ssem = rest[2 * _NBUF:]
        wid = lax.axis_index("s") * _NC + lax.axis_index("c")
        base = wid * bpw
        pltpu.sync_copy(idx_hbm.at[wid], idx_v)

        for c in range(_AHEAD):
            pltpu.async_copy(table_hbm.at[idx_v.at[c]], bufs[c], gsem[c])

        @pl.loop(0, nchunk, step=_NBUF)
        def _(g):
            for u in range(_NBUF):
                c = g + u
                bj = (u + _AHEAD) % _NBUF
                j = c + _AHEAD

                @pl.when(j < nchunk)
                def _():
                    @pl.when(j >= _NBUF)
                    def _():
                        pltpu.make_async_copy(
                            bufs[bj],
                            out_hbm.at[pl.ds(base + (j - _NBUF) * _C, _C)],
                            ssem[bj]).wait()

                    pltpu.async_copy(
                        table_hbm.at[idx_v.at[j]], bufs[bj], gsem[bj])

                pltpu.make_async_copy(
                    table_hbm.at[idx_v.at[c]], bufs[u], gsem[u]).wait()
                pltpu.async_copy(
                    bufs[u], out_hbm.at[pl.ds(base + c * _C, _C)], ssem[u])

        for u in range(_NBUF):
            c_last = nchunk - _NBUF + u
            pltpu.make_async_copy(
                bufs[u], out_hbm.at[pl.ds(base + c_last * _C, _C)],
                ssem[u]).wait()

    return lookup


def kernel(input_ids, weight):
    orig_shape = input_ids.shape
    d = weight.shape[1]
    w32 = weight.astype(jnp.float32)
    if input_ids.dtype != jnp.int32:
        input_ids = input_ids.astype(jnp.int32)

    if len(orig_shape) == 2:
        r, k = orig_shape
        npart = len(_col_parts(k))
        if (r % _NW == 0 and k % 8 == 0 and _NBUF % npart == 0
                and ((r // _NW) * npart) % _NBUF == 0):
            out = _make_lookup_2d(r, k, d)(input_ids, w32)
            return jnp.reshape(out, orig_shape + (d,))

    flat = jnp.reshape(input_ids, (-1,))
    b = flat.shape[0]
    blk = _NW * _C * _NBUF
    b_pad = ((b + blk - 1) // blk) * blk
    if b_pad != b:
        flat = jnp.concatenate([flat, jnp.zeros((b_pad - b,), jnp.int32)])
    idx = jnp.reshape(flat, (_NW, b_pad // (_NW * _C), _C))
    out = _make_lookup_flat(b_pad, d)(w32, idx)
    if b_pad != b:
        out = out[:b]
    return jnp.reshape(out, orig_shape + (d,))


# speedup vs baseline: 1.8295x; 1.0001x over previous
import functools

import jax
import jax.numpy as jnp
from jax import lax
from jax.experimental import pallas as pl
from jax.experimental.pallas import tpu as pltpu
from jax.experimental.pallas import tpu_sc as plsc

_NC = 2
_NS = 16
_NW = _NC * _NS
_C = 128
_NBUF = 8
_AHEAD = 6


def _col_parts(k: int):
    return [(off, min(_C, k - off)) for off in range(0, k, _C)]


@functools.lru_cache(maxsize=None)
def _make_lookup_2d(r_total: int, k: int, d: int):
    rw = r_total // _NW
    parts = _col_parts(k)
    npart = len(parts)
    units = rw * npart
    mesh = plsc.VectorSubcoreMesh(
        core_axis_name="c", subcore_axis_name="s",
        num_cores=_NC, num_subcores=_NS,
    )

    @functools.partial(
        pl.kernel,
        out_type=jax.ShapeDtypeStruct((r_total * k, d), jnp.float32),
        mesh=mesh,
        scratch_types=[pltpu.VMEM((rw, k), jnp.int32)]
        + [pltpu.VMEM((parts[u % npart][1], d), jnp.float32)
           for u in range(_NBUF)]
        + [pltpu.SemaphoreType.DMA] * (2 * _NBUF),
    )
    def lookup(ids_hbm, table_hbm, out_hbm, idx_v, *rest):
        bufs = rest[:_NBUF]
        gsem = rest[_NBUF:2 * _NBUF]
        ssem = rest[2 * _NBUF:]
        wid = lax.axis_index("s") * _NC + lax.axis_index("c")
        base = wid * rw * k
        pltpu.sync_copy(ids_hbm.at[pl.ds(wid * rw, rw)], idx_v)

        def gidx(u_off, g):
            off, w = parts[u_off % npart]
            return idx_v.at[(g + u_off) // npart, pl.ds(off, w)]

        def oslice(u_off, g):
            off, w = parts[u_off % npart]
            return out_hbm.at[
                pl.ds(base + ((g + u_off) // npart) * k + off, w)]

        for u in range(_AHEAD):
            pltpu.async_copy(table_hbm.at[gidx(u, 0)], bufs[u], gsem[u])

        @pl.loop(0, units, step=_NBUF)
        def _(g):
            for u in range(_NBUF):
                uj = u + _AHEAD
                bj = uj % _NBUF
                ju = g + uj

                @pl.when(ju < units)
                def _():
                    @pl.when(ju >= _NBUF)
                    def _():
                        pltpu.make_async_copy(
                            bufs[bj], oslice(uj - _NBUF, g), ssem[bj]).wait()

                    pltpu.async_copy(
                        table_hbm.at[gidx(uj, g)], bufs[bj], gsem[bj])

                pltpu.make_async_copy(
                    table_hbm.at[gidx(u, g)], bufs[u], gsem[u]).wait()
                pltpu.async_copy(bufs[u], oslice(u, g), ssem[u])

        for u in range(_NBUF):
            pltpu.make_async_copy(
                bufs[u], oslice(u, units - _NBUF), ssem[u]).wait()

    return lookup


@functools.lru_cache(maxsize=None)
def _make_lookup_flat(b_total: int, d: int):
    bpw = b_total // _NW
    nchunk = bpw // _C
    mesh = plsc.VectorSubcoreMesh(
        core_axis_name="c", subcore_axis_name="s",
        num_cores=_NC, num_subcores=_NS,
    )

    @functools.partial(
        pl.kernel,
        out_type=jax.ShapeDtypeStruct((b_total, d), jnp.float32),
        mesh=mesh,
        scratch_types=[pltpu.VMEM((nchunk, _C), jnp.int32)]
        + [pltpu.VMEM((_C, d), jnp.float32)] * _NBUF
        + [pltpu.SemaphoreType.DMA] * (2 * _NBUF),
    )
    def lookup(table_hbm, idx_hbm, out_hbm, idx_v, *rest):
        bufs = rest[:_NBUF]
        gsem = rest[_NBUF:2 * _NBUF]
        ssem = rest[2 * _NBUF:]
        wid = lax.axis_index("s") * _NC + lax.axis_index("c")
        base = wid * bpw
        pltpu.sync_copy(idx_hbm.at[wid], idx_v)

        for c in range(_AHEAD):
            pltpu.async_copy(table_hbm.at[idx_v.at[c]], bufs[c], gsem[c])

        @pl.loop(0, nchunk, step=_NBUF)
        def _(g):
            for u in range(_NBUF):
                c = g + u
                bj = (u + _AHEAD) % _NBUF
                j = c + _AHEAD

                @pl.when(j < nchunk)
                def _():
                    @pl.when(j >= _NBUF)
                    def _():
                        pltpu.make_async_copy(
                            bufs[bj],
                            out_hbm.at[pl.ds(base + (j - _NBUF) * _C, _C)],
                            ssem[bj]).wait()

                    pltpu.async_copy(
                        table_hbm.at[idx_v.at[j]], bufs[bj], gsem[bj])

                pltpu.make_async_copy(
                    table_hbm.at[idx_v.at[c]], bufs[u], gsem[u]).wait()
                pltpu.async_copy(
                    bufs[u], out_hbm.at[pl.ds(base + c * _C, _C)], ssem[u])

        for u in range(_NBUF):
            c_last = nchunk - _NBUF + u
            pltpu.make_async_copy(
                bufs[u], out_hbm.at[pl.ds(base + c_last * _C, _C)],
                ssem[u]).wait()

    return lookup


def kernel(input_ids, weight):
    orig_shape = input_ids.shape
    d = weight.shape[1]
    w32 = weight.astype(jnp.float32)
    if input_ids.dtype != jnp.int32:
        input_ids = input_ids.astype(jnp.int32)

    if len(orig_shape) == 2:
        r, k = orig_shape
        npart = len(_col_parts(k))
        if (r % _NW == 0 and k % 8 == 0 and _NBUF % npart == 0
                and ((r // _NW) * npart) % _NBUF == 0):
            out = _make_lookup_2d(r, k, d)(input_ids, w32)
            return jnp.reshape(out, orig_shape + (d,))

    flat = jnp.reshape(input_ids, (-1,))
    b = flat.shape[0]
    blk = _NW * _C * _NBUF
    b_pad = ((b + blk - 1) // blk) * blk
    if b_pad != b:
        flat = jnp.concatenate([flat, jnp.zeros((b_pad - b,), jnp.int32)])
    idx = jnp.reshape(flat, (_NW, b_pad // (_NW * _C), _C))
    out = _make_lookup_flat(b_pad, d)(w32, idx)
    if b_pad != b:
        out = out[:b]
    return jnp.reshape(out, orig_shape + (d,))
